# baseline (device time: 72863 ns/iter reference)
import jax
import jax.numpy as jnp
from jax import lax
from jax.experimental import pallas as pl
from jax.experimental.pallas import tpu as pltpu

T = 1024
D = 2048
V_SHARD = 16384
V_DEV = V_SHARD // 2
BLK = 1024
NBLK = V_DEV // BLK
N_DEV = 4


def _fused_body(mx_ref, x_hbm, w_ref, lab_hbm, out_hbm,
                x_ref, lab_ref, acc_ref, comm_ref,
                send_sems, recv_sems, local_sems):
    j = pl.program_id(0)
    mx = lax.axis_index("x")
    my = lax.axis_index("y")
    my_l = 2 * mx + my

    @pl.when(j == 0)
    def _():
        cp_x = pltpu.make_async_copy(x_hbm, x_ref, local_sems.at[0])
        cp_l = pltpu.make_async_copy(lab_hbm, lab_ref, local_sems.at[1])
        cp_x.start()
        cp_l.start()

        barrier = pltpu.get_barrier_semaphore()
        for p in range(N_DEV):
            pl.semaphore_signal(barrier, inc=1, device_id=(p // 2, p % 2),
                                device_id_type=pl.DeviceIdType.MESH)
        pl.semaphore_wait(barrier, N_DEV)

        cp_x.wait()
        cp_l.wait()

        acc_ref[...] = jnp.zeros((8, T), jnp.float32)
        acc_ref[0, :] = jnp.full((T,), -jnp.inf, jnp.float32)

    @pl.when(j < NBLK)
    def _():
        logits = jnp.dot(x_ref[...], w_ref[...],
                         preferred_element_type=jnp.float32)
        m_prev = acc_ref[0, :]
        s_prev = acc_ref[1, :]
        g_prev = acc_ref[2, :]

        bm = jnp.max(logits, axis=1)
        m_new = jnp.maximum(m_prev, bm)
        s_new = s_prev * jnp.exp(m_prev - m_new) + jnp.sum(
            jnp.exp(logits - m_new[:, None]), axis=1)

        col0 = my * V_SHARD + mx * V_DEV + j * BLK
        cols = col0 + lax.broadcasted_iota(jnp.int32, (T, BLK), 1)
        hit = cols == lab_ref[...][:, None]
        g_new = g_prev + jnp.sum(jnp.where(hit, logits, 0.0), axis=1)

        acc_ref[0, :] = m_new
        acc_ref[1, :] = s_new
        acc_ref[2, :] = g_new

    @pl.when(j == NBLK)
    def _():
        for p in range(N_DEV):
            pltpu.make_async_remote_copy(
                src_ref=acc_ref,
                dst_ref=comm_ref.at[my_l],
                send_sem=send_sems.at[p],
                recv_sem=recv_sems.at[my_l],
                device_id=(p // 2, p % 2),
                device_id_type=pl.DeviceIdType.MESH,
            ).start()
        for p in range(N_DEV):
            r = pltpu.make_async_remote_copy(
                src_ref=acc_ref,
                dst_ref=comm_ref.at[p],
                send_sem=send_sems.at[p],
                recv_sem=recv_sems.at[p],
                device_id=(p // 2, p % 2),
                device_id_type=pl.DeviceIdType.MESH,
            )
            r.wait_send()
            r.wait_recv()

        m = comm_ref[0, 0, :]
        for k in range(1, N_DEV):
            m = jnp.maximum(m, comm_ref[k, 0, :])
        s = jnp.zeros((T,), jnp.float32)
        g = jnp.zeros((T,), jnp.float32)
        for k in range(N_DEV):
            s = s + comm_ref[k, 1, :] * jnp.exp(comm_ref[k, 0, :] - m)
            g = g + comm_ref[k, 2, :]
        acc_ref[3, :] = m + jnp.log(s) - g

        cp_out = pltpu.make_async_copy(acc_ref.at[3], out_hbm,
                                       local_sems.at[2])
        cp_out.start()
        cp_out.wait()


def kernel(x, W, labels):
    mx_arr = jnp.reshape(lax.axis_index("x"), (1,))
    x = pltpu.with_memory_space_constraint(x, pltpu.MemorySpace.HBM)
    labels = pltpu.with_memory_space_constraint(labels, pltpu.MemorySpace.HBM)

    grid_spec = pltpu.PrefetchScalarGridSpec(
        num_scalar_prefetch=1,
        grid=(NBLK + 1,),
        in_specs=[
            pl.BlockSpec(memory_space=pl.ANY),
            pl.BlockSpec(
                (D, BLK),
                lambda j, mxr: (0, mxr[0] * NBLK + jnp.minimum(j, NBLK - 1)),
            ),
            pl.BlockSpec(memory_space=pl.ANY),
        ],
        out_specs=pl.BlockSpec(memory_space=pl.ANY),
        scratch_shapes=[
            pltpu.VMEM((T, D), jnp.float32),
            pltpu.VMEM((T,), jnp.int32),
            pltpu.VMEM((8, T), jnp.float32),
            pltpu.VMEM((N_DEV, 8, T), jnp.float32),
            pltpu.SemaphoreType.DMA((N_DEV,)),
            pltpu.SemaphoreType.DMA((N_DEV,)),
            pltpu.SemaphoreType.DMA((3,)),
        ],
    )
    return pl.pallas_call(
        _fused_body,
        grid_spec=grid_spec,
        out_shape=jax.ShapeDtypeStruct((T,), jnp.float32),
        compiler_params=pltpu.CompilerParams(
            dimension_semantics=("arbitrary",),
            collective_id=0,
        ),
    )(mx_arr, x, W, labels)


# device time: 72797 ns/iter; 1.0009x vs baseline; 1.0009x over previous
import jax
import jax.numpy as jnp
from jax import lax
from jax.experimental import pallas as pl
from jax.experimental.pallas import tpu as pltpu

T = 1024
D = 2048
V_SHARD = 16384
V_DEV = V_SHARD // 2
BLK = 1024
NBLK = V_DEV // BLK
N_DEV = 4


def _fused_body(mx_ref, x_hbm, w_ref, lab_hbm, out_hbm,
                x_ref, lab_ref, acc_ref, comm_ref,
                send_sems, recv_sems, local_sems):
    j = pl.program_id(0)
    mx = lax.axis_index("x")
    my = lax.axis_index("y")
    my_l = 2 * mx + my

    @pl.when(j == 0)
    def _():
        n_chunk = 4
        rows = T // n_chunk
        cps = [
            pltpu.make_async_copy(
                x_hbm.at[pl.ds(c * rows, rows)],
                x_ref.at[pl.ds(c * rows, rows)],
                local_sems.at[c],
            )
            for c in range(n_chunk)
        ]
        cp_l = pltpu.make_async_copy(lab_hbm, lab_ref,
                                     local_sems.at[n_chunk])
        for cp in cps:
            cp.start()
        cp_l.start()

        barrier = pltpu.get_barrier_semaphore()
        for p in range(N_DEV):
            pl.semaphore_signal(barrier, inc=1, device_id=(p // 2, p % 2),
                                device_id_type=pl.DeviceIdType.MESH)
        pl.semaphore_wait(barrier, N_DEV)

        for cp in cps:
            cp.wait()
        cp_l.wait()

        acc_ref[...] = jnp.zeros((8, T), jnp.float32)
        acc_ref[0, :] = jnp.full((T,), -jnp.inf, jnp.float32)

    @pl.when(j < NBLK)
    def _():
        logits = jnp.dot(x_ref[...], w_ref[...],
                         preferred_element_type=jnp.float32)
        m_prev = acc_ref[0, :]
        s_prev = acc_ref[1, :]
        g_prev = acc_ref[2, :]

        bm = jnp.max(logits, axis=1)
        m_new = jnp.maximum(m_prev, bm)
        s_new = s_prev * jnp.exp(m_prev - m_new) + jnp.sum(
            jnp.exp(logits - m_new[:, None]), axis=1)

        col0 = my * V_SHARD + mx * V_DEV + j * BLK
        cols = col0 + lax.broadcasted_iota(jnp.int32, (T, BLK), 1)
        hit = cols == lab_ref[...][:, None]
        g_new = g_prev + jnp.sum(jnp.where(hit, logits, 0.0), axis=1)

        acc_ref[0, :] = m_new
        acc_ref[1, :] = s_new
        acc_ref[2, :] = g_new

    @pl.when(j == NBLK)
    def _():
        for p in range(N_DEV):
            pltpu.make_async_remote_copy(
                src_ref=acc_ref,
                dst_ref=comm_ref.at[my_l],
                send_sem=send_sems.at[p],
                recv_sem=recv_sems.at[my_l],
                device_id=(p // 2, p % 2),
                device_id_type=pl.DeviceIdType.MESH,
            ).start()
        for p in range(N_DEV):
            r = pltpu.make_async_remote_copy(
                src_ref=acc_ref,
                dst_ref=comm_ref.at[p],
                send_sem=send_sems.at[p],
                recv_sem=recv_sems.at[p],
                device_id=(p // 2, p % 2),
                device_id_type=pl.DeviceIdType.MESH,
            )
            r.wait_send()
            r.wait_recv()

        m = comm_ref[0, 0, :]
        for k in range(1, N_DEV):
            m = jnp.maximum(m, comm_ref[k, 0, :])
        s = jnp.zeros((T,), jnp.float32)
        g = jnp.zeros((T,), jnp.float32)
        for k in range(N_DEV):
            s = s + comm_ref[k, 1, :] * jnp.exp(comm_ref[k, 0, :] - m)
            g = g + comm_ref[k, 2, :]
        acc_ref[3, :] = m + jnp.log(s) - g

        cp_out = pltpu.make_async_copy(acc_ref.at[3], out_hbm,
                                       local_sems.at[5])
        cp_out.start()
        cp_out.wait()


def kernel(x, W, labels):
    mx_arr = jnp.reshape(lax.axis_index("x"), (1,))
    x = pltpu.with_memory_space_constraint(x, pltpu.MemorySpace.HBM)
    labels = pltpu.with_memory_space_constraint(labels, pltpu.MemorySpace.HBM)

    grid_spec = pltpu.PrefetchScalarGridSpec(
        num_scalar_prefetch=1,
        grid=(NBLK + 1,),
        in_specs=[
            pl.BlockSpec(memory_space=pl.ANY),
            pl.BlockSpec(
                (D, BLK),
                lambda j, mxr: (0, mxr[0] * NBLK + jnp.minimum(j, NBLK - 1)),
            ),
            pl.BlockSpec(memory_space=pl.ANY),
        ],
        out_specs=pl.BlockSpec(memory_space=pl.ANY),
        scratch_shapes=[
            pltpu.VMEM((T, D), jnp.float32),
            pltpu.VMEM((T,), jnp.int32),
            pltpu.VMEM((8, T), jnp.float32),
            pltpu.VMEM((N_DEV, 8, T), jnp.float32),
            pltpu.SemaphoreType.DMA((N_DEV,)),
            pltpu.SemaphoreType.DMA((N_DEV,)),
            pltpu.SemaphoreType.DMA((6,)),
        ],
    )
    return pl.pallas_call(
        _fused_body,
        grid_spec=grid_spec,
        out_shape=jax.ShapeDtypeStruct((T,), jnp.float32),
        compiler_params=pltpu.CompilerParams(
            dimension_semantics=("arbitrary",),
            collective_id=0,
        ),
    )(mx_arr, x, W, labels)


# device time: 71776 ns/iter; 1.0151x vs baseline; 1.0142x over previous
import jax
import jax.numpy as jnp
from jax import lax
from jax.experimental import pallas as pl
from jax.experimental.pallas import tpu as pltpu

T = 1024
D = 2048
V_SHARD = 16384
V_DEV = V_SHARD // 2
BLK = 1024
NBLK = V_DEV // BLK
N_DEV = 4
N_XCHUNK = 4
XROWS = T // N_XCHUNK


def _update(acc_ref, lab_ref, logits, mx, my, j):
    m_prev = acc_ref[0, :]
    s_prev = acc_ref[1, :]
    g_prev = acc_ref[2, :]

    bm = jnp.max(logits, axis=1)
    m_new = jnp.maximum(m_prev, bm)
    s_new = s_prev * jnp.exp(m_prev - m_new) + jnp.sum(
        jnp.exp(logits - m_new[:, None]), axis=1)

    col0 = my * V_SHARD + mx * V_DEV + j * BLK
    cols = col0 + lax.broadcasted_iota(jnp.int32, (T, BLK), 1)
    hit = cols == lab_ref[...][:, None]
    g_new = g_prev + jnp.sum(jnp.where(hit, logits, 0.0), axis=1)

    acc_ref[0, :] = m_new
    acc_ref[1, :] = s_new
    acc_ref[2, :] = g_new


def _fused_body(mx_ref, x_hbm, w_ref, lab_hbm, out_hbm,
                x_ref, lab_ref, acc_ref, comm_ref,
                send_sems, recv_sems, local_sems):
    j = pl.program_id(0)
    mx = lax.axis_index("x")
    my = lax.axis_index("y")
    my_l = 2 * mx + my

    @pl.when(j == 0)
    def _():
        cps = [
            pltpu.make_async_copy(
                x_hbm.at[pl.ds(c * XROWS, XROWS)],
                x_ref.at[pl.ds(c * XROWS, XROWS)],
                local_sems.at[c],
            )
            for c in range(N_XCHUNK)
        ]
        cp_l = pltpu.make_async_copy(lab_hbm, lab_ref,
                                     local_sems.at[N_XCHUNK])
        for cp in cps:
            cp.start()
        cp_l.start()

        barrier = pltpu.get_barrier_semaphore()
        for p in range(N_DEV):
            pl.semaphore_signal(barrier, inc=1, device_id=(p // 2, p % 2),
                                device_id_type=pl.DeviceIdType.MESH)
        pl.semaphore_wait(barrier, N_DEV)

        parts = []
        for c in range(N_XCHUNK):
            cps[c].wait()
            parts.append(jnp.dot(x_ref[pl.ds(c * XROWS, XROWS), :],
                                 w_ref[...],
                                 preferred_element_type=jnp.float32))
        logits = jnp.concatenate(parts, axis=0)
        cp_l.wait()

        acc_ref[...] = jnp.zeros((4, T), jnp.float32)
        acc_ref[0, :] = jnp.full((T,), -jnp.inf, jnp.float32)
        _update(acc_ref, lab_ref, logits, mx, my, j)

    @pl.when((j > 0) & (j < NBLK))
    def _():
        logits = jnp.dot(x_ref[...], w_ref[...],
                         preferred_element_type=jnp.float32)
        _update(acc_ref, lab_ref, logits, mx, my, j)

    @pl.when(j == NBLK)
    def _():
        for p in range(N_DEV):
            pltpu.make_async_remote_copy(
                src_ref=acc_ref,
                dst_ref=comm_ref.at[my_l],
                send_sem=send_sems.at[p],
                recv_sem=recv_sems.at[my_l],
                device_id=(p // 2, p % 2),
                device_id_type=pl.DeviceIdType.MESH,
            ).start()
        for p in range(N_DEV):
            r = pltpu.make_async_remote_copy(
                src_ref=acc_ref,
                dst_ref=comm_ref.at[p],
                send_sem=send_sems.at[p],
                recv_sem=recv_sems.at[p],
                device_id=(p // 2, p % 2),
                device_id_type=pl.DeviceIdType.MESH,
            )
            r.wait_send()
            r.wait_recv()

        m = comm_ref[0, 0, :]
        for k in range(1, N_DEV):
            m = jnp.maximum(m, comm_ref[k, 0, :])
        s = jnp.zeros((T,), jnp.float32)
        g = jnp.zeros((T,), jnp.float32)
        for k in range(N_DEV):
            s = s + comm_ref[k, 1, :] * jnp.exp(comm_ref[k, 0, :] - m)
            g = g + comm_ref[k, 2, :]
        acc_ref[3, :] = m + jnp.log(s) - g

        cp_out = pltpu.make_async_copy(acc_ref.at[3], out_hbm,
                                       local_sems.at[N_XCHUNK + 1])
        cp_out.start()
        cp_out.wait()


def kernel(x, W, labels):
    mx_arr = jnp.reshape(lax.axis_index("x"), (1,))
    x = pltpu.with_memory_space_constraint(x, pltpu.MemorySpace.HBM)
    labels = pltpu.with_memory_space_constraint(labels, pltpu.MemorySpace.HBM)

    grid_spec = pltpu.PrefetchScalarGridSpec(
        num_scalar_prefetch=1,
        grid=(NBLK + 1,),
        in_specs=[
            pl.BlockSpec(memory_space=pl.ANY),
            pl.BlockSpec(
                (D, BLK),
                lambda j, mxr: (0, mxr[0] * NBLK + jnp.minimum(j, NBLK - 1)),
            ),
            pl.BlockSpec(memory_space=pl.ANY),
        ],
        out_specs=pl.BlockSpec(memory_space=pl.ANY),
        scratch_shapes=[
            pltpu.VMEM((T, D), jnp.float32),
            pltpu.VMEM((T,), jnp.int32),
            pltpu.VMEM((4, T), jnp.float32),
            pltpu.VMEM((N_DEV, 4, T), jnp.float32),
            pltpu.SemaphoreType.DMA((N_DEV,)),
            pltpu.SemaphoreType.DMA((N_DEV,)),
            pltpu.SemaphoreType.DMA((N_XCHUNK + 2,)),
        ],
    )
    return pl.pallas_call(
        _fused_body,
        grid_spec=grid_spec,
        out_shape=jax.ShapeDtypeStruct((T,), jnp.float32),
        compiler_params=pltpu.CompilerParams(
            dimension_semantics=("arbitrary",),
            collective_id=0,
        ),
    )(mx_arr, x, W, labels)
